# baseline (speedup 1.0000x reference)
import jax
import jax.numpy as jnp
from jax import lax
from jax.experimental import pallas as pl
from jax.experimental.pallas import tpu as pltpu
from jax.experimental.pallas import tpu_sc as plsc

B = 16384
ED = 64
OUT_D = 3 * ED
SV, VV, TV = 3, 4, 3
NCOMBO = SV * VV * TV

NC, NS, L = 2, 16, 16
NW = NC * NS
BPW = B // NW
RB = BPW // 128
CB = OUT_D // 8
NFLAT = B * OUT_D // 128


def _body(sess_hbm, vol_hbm, trend_hbm, fused_hbm, out_hbm,
          sidx_v, vidx_v, tidx_v, table_v, buf_v, sem):
    wid = lax.axis_index("s") * NC + lax.axis_index("c")
    base = wid * BPW

    in_copies = [
        pltpu.async_copy(sess_hbm.at[pl.ds(base, BPW)], sidx_v, sem),
        pltpu.async_copy(vol_hbm.at[pl.ds(base, BPW)], vidx_v, sem),
        pltpu.async_copy(trend_hbm.at[pl.ds(base, BPW)], tidx_v, sem),
        pltpu.async_copy(fused_hbm, table_v, sem),
    ]
    for cp in in_copies:
        cp.wait()

    for r in range(RB):
        combos = []
        for m in range(8):
            o = r * 128 + m * L
            s = sidx_v[pl.ds(o, L)]
            v = vidx_v[pl.ds(o, L)]
            t = tidx_v[pl.ds(o, L)]
            s = jnp.minimum(jnp.maximum(s, 0), SV - 1)
            v = jnp.minimum(jnp.maximum(v, 0), VV - 1)
            t = jnp.minimum(jnp.maximum(t, 0), TV - 1)
            combos.append((s * (VV * TV) + v * TV + t) * (OUT_D + 1))

        @plsc.parallel_loop(0, CB, 1, unroll=8)
        def cbody(c, _combos=combos, _r=r):
            for f in range(8):
                d = c * 8 + f
                row = c * (RB * 8) + _r * 8 + f
                for m in range(8):
                    x = plsc.load_gather(table_v, [_combos[m] + d])
                    buf_v[row, pl.ds(m * L, L)] = x

    copies = [
        pltpu.async_copy(
            buf_v.at[pl.ds(c * (RB * 8), RB * 8)],
            out_hbm.at[pl.ds(c * (B // 128) * 8 + wid * (RB * 8), RB * 8)],
            sem,
        )
        for c in range(CB)
    ]
    for cp in copies:
        cp.wait()


def kernel(session_id, vol_regime_id, trend_regime_id,
           session_table, vol_table, trend_table):
    c = jnp.arange(NCOMBO, dtype=jnp.int32)
    fused = jnp.concatenate(
        [
            jnp.take(session_table, c // (VV * TV), axis=0),
            jnp.take(vol_table, (c // TV) % VV, axis=0),
            jnp.take(trend_table, c % TV, axis=0),
        ],
        axis=-1,
    )

    run = pl.kernel(
        _body,
        mesh=plsc.VectorSubcoreMesh(core_axis_name="c", subcore_axis_name="s"),
        out_type=jax.ShapeDtypeStruct((NFLAT, 128), jnp.float32),
        scratch_types=[
            pltpu.VMEM((BPW,), jnp.int32),
            pltpu.VMEM((BPW,), jnp.int32),
            pltpu.VMEM((BPW,), jnp.int32),
            pltpu.VMEM((NCOMBO * (OUT_D + 1),), jnp.float32),
            pltpu.VMEM((CB * RB * 8, 128), jnp.float32),
            pltpu.SemaphoreType.DMA,
        ],
        compiler_params=pltpu.CompilerParams(
            use_tc_tiling_on_sc=False, needs_layout_passes=False),
    )
    flat = run(
        session_id.astype(jnp.int32),
        vol_regime_id.astype(jnp.int32),
        trend_regime_id.astype(jnp.int32),
        jnp.pad(fused, ((0, 0), (0, 1))).reshape(-1),
    )
    return (flat.reshape(CB, B // 128, 8, 128)
            .transpose(1, 3, 0, 2)
            .reshape(B, OUT_D))


# speedup vs baseline: 1.1269x; 1.1269x over previous
import jax
import jax.numpy as jnp
from jax import lax
from jax.experimental import pallas as pl
from jax.experimental.pallas import tpu as pltpu
from jax.experimental.pallas import tpu_sc as plsc

B = 16384
ED = 64
OUT_D = 3 * ED
SV, VV, TV = 3, 4, 3
NCOMBO = SV * VV * TV

NC, NS, L = 2, 16, 16
NW = NC * NS
BPW = B // NW
RB = BPW // 128
CB = OUT_D // 8
NFLAT = B * OUT_D // 128


def _body(sess_hbm, vol_hbm, trend_hbm, fused_hbm, out_hbm,
          sidx_v, vidx_v, tidx_v, table_v, buf_v, sem):
    wid = lax.axis_index("s") * NC + lax.axis_index("c")
    base = wid * BPW

    in_copies = [
        pltpu.async_copy(sess_hbm.at[pl.ds(base, BPW)], sidx_v, sem),
        pltpu.async_copy(vol_hbm.at[pl.ds(base, BPW)], vidx_v, sem),
        pltpu.async_copy(trend_hbm.at[pl.ds(base, BPW)], tidx_v, sem),
        pltpu.async_copy(fused_hbm, table_v, sem),
    ]
    for cp in in_copies:
        cp.wait()

    for r in range(RB):
        combos = []
        for m in range(8):
            o = r * 128 + m * L
            s = sidx_v[pl.ds(o, L)]
            v = vidx_v[pl.ds(o, L)]
            t = tidx_v[pl.ds(o, L)]
            s = jnp.minimum(jnp.maximum(s, 0), SV - 1)
            v = jnp.minimum(jnp.maximum(v, 0), VV - 1)
            t = jnp.minimum(jnp.maximum(t, 0), TV - 1)
            combos.append((s * (VV * TV) + v * TV + t) * (OUT_D + 1))

        @plsc.parallel_loop(0, CB, 1, unroll=4)
        def cbody(c, _combos=combos, _r=r):
            for f in range(8):
                d = c * 8 + f
                row = c * (RB * 8) + _r * 8 + f
                for m in range(8):
                    x = plsc.load_gather(table_v, [_combos[m] + d])
                    buf_v[row, pl.ds(m * L, L)] = x

    copies = [
        pltpu.async_copy(
            buf_v.at[pl.ds(c * (RB * 8), RB * 8)],
            out_hbm.at[pl.ds(c * (B // 128) * 8 + wid * (RB * 8), RB * 8)],
            sem,
        )
        for c in range(CB)
    ]
    for cp in copies:
        cp.wait()


def kernel(session_id, vol_regime_id, trend_regime_id,
           session_table, vol_table, trend_table):
    c = jnp.arange(NCOMBO, dtype=jnp.int32)
    fused = jnp.concatenate(
        [
            jnp.take(session_table, c // (VV * TV), axis=0),
            jnp.take(vol_table, (c // TV) % VV, axis=0),
            jnp.take(trend_table, c % TV, axis=0),
        ],
        axis=-1,
    )

    run = pl.kernel(
        _body,
        mesh=plsc.VectorSubcoreMesh(core_axis_name="c", subcore_axis_name="s"),
        out_type=jax.ShapeDtypeStruct((NFLAT, 128), jnp.float32),
        scratch_types=[
            pltpu.VMEM((BPW,), jnp.int32),
            pltpu.VMEM((BPW,), jnp.int32),
            pltpu.VMEM((BPW,), jnp.int32),
            pltpu.VMEM((NCOMBO * (OUT_D + 1),), jnp.float32),
            pltpu.VMEM((CB * RB * 8, 128), jnp.float32),
            pltpu.SemaphoreType.DMA,
        ],
        compiler_params=pltpu.CompilerParams(
            use_tc_tiling_on_sc=False, needs_layout_passes=False),
    )
    flat = run(
        session_id.astype(jnp.int32),
        vol_regime_id.astype(jnp.int32),
        trend_regime_id.astype(jnp.int32),
        jnp.pad(fused, ((0, 0), (0, 1))).reshape(-1),
    )
    return (flat.reshape(CB, B // 128, 8, 128)
            .transpose(1, 3, 0, 2)
            .reshape(B, OUT_D))
